# baseline (speedup 1.0000x reference)
import functools

import jax
import jax.numpy as jnp
from jax import lax
from jax.experimental import pallas as pl
from jax.experimental.pallas import tpu as pltpu
from jax.experimental.pallas import tpu_sc as plsc

_V = 1_000_000
_D = 64
_R = 16384
_C = 20
_NW = 32
_ROWS_PER_W = _R // _NW
_BLK_ROWS = 32
_NCHUNK = _ROWS_PER_W // _BLK_ROWS
_RING = 2
_NSUPER = _NCHUNK // _RING

_mesh = plsc.VectorSubcoreMesh(core_axis_name="c", subcore_axis_name="s")


@functools.partial(
    pl.kernel,
    out_type=jax.ShapeDtypeStruct((_R, _C, _D), jnp.float32),
    mesh=_mesh,
    scratch_types=[
        pltpu.VMEM((_ROWS_PER_W, _C), jnp.int32),
        pltpu.VMEM((_RING * _BLK_ROWS, _C, _D), jnp.float32),
        pltpu.SemaphoreType.DMA,
        pltpu.SemaphoreType.DMA,
        pltpu.SemaphoreType.DMA,
    ],
    compiler_params=pltpu.CompilerParams(use_tc_tiling_on_sc=False),
)
def _embed_gather(x_hbm, w_hbm, out_hbm, idx_v, rows_v, sem_g,
                  sem_w0, sem_w1):
    sem_w = (sem_w0, sem_w1)
    wid = lax.axis_index("s") * 2 + lax.axis_index("c")
    row0 = pl.multiple_of(wid * _ROWS_PER_W, _ROWS_PER_W)

    pltpu.sync_copy(x_hbm.at[pl.ds(row0, _ROWS_PER_W), :], idx_v)

    def dbl_row(i, _):
        a = idx_v[i, pl.ds(0, 16)]
        b = idx_v[i, pl.ds(4, 16)]
        idx_v[i, pl.ds(0, 16)] = a * 2
        idx_v[i, pl.ds(4, 16)] = b * 2
        return ()

    lax.fori_loop(0, _ROWS_PER_W, dbl_row, ())

    def fire_gathers(k, p):
        def fire_row(r, _):
            pltpu.async_copy(
                w_hbm.at[idx_v.at[k * _BLK_ROWS + r, :]],
                rows_v.at[p * _BLK_ROWS + r, :, :],
                sem_g,
            )
            return ()
        lax.fori_loop(0, _BLK_ROWS, fire_row, ())

    for p in range(_RING):
        fire_gathers(p, p)

    def super_block(g, _):
        for p in range(_RING):
            k = g * _RING + p
            blk_row = row0 + k * _BLK_ROWS
            slot = rows_v.at[pl.ds(p * _BLK_ROWS, _BLK_ROWS), :, :]
            pltpu.make_async_copy(
                out_hbm.at[pl.ds(0, _BLK_ROWS), :, :], slot, sem_g
            ).wait()
            pltpu.async_copy(
                slot, out_hbm.at[pl.ds(blk_row, _BLK_ROWS), :, :], sem_w[p]
            )

            @pl.when(g < _NSUPER - 1)
            def _():
                pltpu.make_async_copy(
                    slot, out_hbm.at[pl.ds(0, _BLK_ROWS), :, :], sem_w[p]
                ).wait()
                fire_gathers(k + _RING, p)

        return ()

    lax.fori_loop(0, _NSUPER, super_block, ())

    for p in range(_RING):
        pltpu.make_async_copy(
            rows_v.at[pl.ds(p * _BLK_ROWS, _BLK_ROWS), :, :],
            out_hbm.at[pl.ds(0, _BLK_ROWS), :, :],
            sem_w[p],
        ).wait()


def kernel(x, weight):
    w2 = jnp.repeat(weight, 2, axis=0)
    return _embed_gather(x, w2)


# speedup vs baseline: 1.9651x; 1.9651x over previous
import functools

import jax
import jax.numpy as jnp
from jax import lax
from jax.experimental import pallas as pl
from jax.experimental.pallas import tpu as pltpu
from jax.experimental.pallas import tpu_sc as plsc

_V = 1_000_000
_D = 64
_R = 16384
_C = 20
_NW = 32
_ROWS_PER_W = _R // _NW
_BLK_ROWS = 32
_NCHUNK = _ROWS_PER_W // _BLK_ROWS
_RING = 2
_NSUPER = _NCHUNK // _RING

_mesh = plsc.VectorSubcoreMesh(core_axis_name="c", subcore_axis_name="s")


@functools.partial(
    pl.kernel,
    out_type=jax.ShapeDtypeStruct((_R, _C, _D), jnp.float32),
    mesh=_mesh,
    scratch_types=[
        pltpu.VMEM((_ROWS_PER_W, _C), jnp.int32),
        pltpu.VMEM((_RING * _BLK_ROWS, _C, _D), jnp.float32),
        pltpu.SemaphoreType.DMA,
        pltpu.SemaphoreType.DMA,
        pltpu.SemaphoreType.DMA,
    ],
    compiler_params=pltpu.CompilerParams(use_tc_tiling_on_sc=False),
)
def _embed_gather(x_hbm, w_hbm, out_hbm, idx_v, rows_v, sem_g,
                  sem_w0, sem_w1):
    sem_w = (sem_w0, sem_w1)
    wid = lax.axis_index("s") * 2 + lax.axis_index("c")
    row0 = pl.multiple_of(wid * _ROWS_PER_W, _ROWS_PER_W)

    pltpu.sync_copy(x_hbm.at[pl.ds(row0, _ROWS_PER_W), :], idx_v)

    def dbl_row(i, _):
        a = idx_v[i, pl.ds(0, 16)]
        b = idx_v[i, pl.ds(4, 16)]
        idx_v[i, pl.ds(0, 16)] = a * 2
        idx_v[i, pl.ds(4, 16)] = b * 2
        return ()

    lax.fori_loop(0, _ROWS_PER_W, dbl_row, ())

    def fire_gathers(k, p):
        def fire_row(r, _):
            pltpu.async_copy(
                w_hbm.at[idx_v.at[k * _BLK_ROWS + r, :]],
                rows_v.at[p * _BLK_ROWS + r, :, :],
                sem_g,
            )
            return ()
        lax.fori_loop(0, _BLK_ROWS, fire_row, ())

    for p in range(_RING):
        fire_gathers(p, p)

    def super_block(g, _):
        for p in range(_RING):
            k = g * _RING + p
            blk_row = row0 + k * _BLK_ROWS
            slot = rows_v.at[pl.ds(p * _BLK_ROWS, _BLK_ROWS), :, :]
            pltpu.make_async_copy(
                out_hbm.at[pl.ds(0, _BLK_ROWS), :, :], slot, sem_g
            ).wait()
            pltpu.async_copy(
                slot, out_hbm.at[pl.ds(blk_row, _BLK_ROWS), :, :], sem_w[p]
            )

            @pl.when(g < _NSUPER - 1)
            def _():
                pltpu.make_async_copy(
                    slot, out_hbm.at[pl.ds(0, _BLK_ROWS), :, :], sem_w[p]
                ).wait()
                fire_gathers(k + _RING, p)

        return ()

    lax.fori_loop(0, _NSUPER, super_block, ())

    for p in range(_RING):
        pltpu.make_async_copy(
            rows_v.at[pl.ds(p * _BLK_ROWS, _BLK_ROWS), :, :],
            out_hbm.at[pl.ds(0, _BLK_ROWS), :, :],
            sem_w[p],
        ).wait()


def kernel(x, weight):
    w2 = jnp.pad(weight.T, ((0, _D), (0, 0))).T.reshape(2 * _V, _D)
    return _embed_gather(x, w2)


# speedup vs baseline: 2.1504x; 1.0943x over previous
import functools

import jax
import jax.numpy as jnp
from jax import lax
from jax.experimental import pallas as pl
from jax.experimental.pallas import tpu as pltpu
from jax.experimental.pallas import tpu_sc as plsc

_V = 1_000_000
_D = 64
_R = 16384
_C = 20
_NW = 32
_ROWS_PER_W = _R // _NW
_BLK_ROWS = 32
_NCHUNK = _ROWS_PER_W // _BLK_ROWS
_RING = 2
_NSUPER = _NCHUNK // _RING

_mesh = plsc.VectorSubcoreMesh(core_axis_name="c", subcore_axis_name="s")


@functools.partial(
    pl.kernel,
    out_type=jax.ShapeDtypeStruct((_R, _C, _D), jnp.float32),
    mesh=_mesh,
    scratch_types=[
        pltpu.VMEM((_ROWS_PER_W, _C), jnp.int32),
        pltpu.VMEM((_RING * _BLK_ROWS, _C, _D), jnp.float32),
        pltpu.SemaphoreType.DMA,
        pltpu.SemaphoreType.DMA,
        pltpu.SemaphoreType.DMA,
    ],
    compiler_params=pltpu.CompilerParams(use_tc_tiling_on_sc=False),
)
def _embed_gather(x_hbm, w_hbm, out_hbm, idx_v, rows_v, sem_g,
                  sem_w0, sem_w1):
    sem_w = (sem_w0, sem_w1)
    wid = lax.axis_index("s") * 2 + lax.axis_index("c")
    row0 = pl.multiple_of(wid * _ROWS_PER_W, _ROWS_PER_W)

    pltpu.sync_copy(x_hbm.at[pl.ds(row0, _ROWS_PER_W), :], idx_v)

    def dbl_row(i, _):
        a = idx_v[i, pl.ds(0, 16)]
        b = idx_v[i, pl.ds(4, 16)]
        idx_v[i, pl.ds(0, 16)] = a * 2
        idx_v[i, pl.ds(4, 16)] = b * 2
        return ()

    lax.fori_loop(0, _ROWS_PER_W, dbl_row, ())

    def fire_gathers(k, p):
        def fire_row(r, _):
            pltpu.async_copy(
                w_hbm.at[idx_v.at[k * _BLK_ROWS + r, :]],
                rows_v.at[p * _BLK_ROWS + r, :, :],
                sem_g,
            )
            return ()
        lax.fori_loop(0, _BLK_ROWS, fire_row, ())

    for p in range(_RING):
        fire_gathers(p, p)

    def super_block(g, _):
        for p in range(_RING):
            k = g * _RING + p
            blk_row = row0 + k * _BLK_ROWS
            slot = rows_v.at[pl.ds(p * _BLK_ROWS, _BLK_ROWS), :, :]
            pltpu.make_async_copy(
                out_hbm.at[pl.ds(0, _BLK_ROWS), :, :], slot, sem_g
            ).wait()
            pltpu.async_copy(
                slot, out_hbm.at[pl.ds(blk_row, _BLK_ROWS), :, :], sem_w[p]
            )

            @pl.when(g < _NSUPER - 1)
            def _():
                pltpu.make_async_copy(
                    slot, out_hbm.at[pl.ds(0, _BLK_ROWS), :, :], sem_w[p]
                ).wait()
                fire_gathers(k + _RING, p)

        return ()

    lax.fori_loop(0, _NSUPER, super_block, ())

    for p in range(_RING):
        pltpu.make_async_copy(
            rows_v.at[pl.ds(p * _BLK_ROWS, _BLK_ROWS), :, :],
            out_hbm.at[pl.ds(0, _BLK_ROWS), :, :],
            sem_w[p],
        ).wait()


def kernel(x, weight):
    w2 = jnp.pad(weight, ((0, 0), (0, _D))).reshape(2 * _V, _D)
    return _embed_gather(x, w2)
